# Initial kernel scaffold; baseline (speedup 1.0000x reference)
import jax
import jax.numpy as jnp
from jax.experimental import pallas as pl


def kernel(x, edge_index, src, dst, Wl1, Wr1, b1, Wl2, Wr2, b2):
    raise NotImplementedError("write your pallas kernel here")


# speedup vs baseline: 4.8274x; 4.8274x over previous
import functools

import jax
import jax.numpy as jnp
from jax import lax
from jax.experimental import pallas as pl
from jax.experimental.pallas import tpu as pltpu
from jax.experimental.pallas import tpu_sc as plsc

N = 10000
E = 320000
D_IN = 128
D_H = 64
NQ = 65536

NC, NS, L = 2, 16, 16
NW = NC * NS
EPW = E // NW
EK = 80
ECH = EPW // EK
QPW = NQ // NW
QK = 128
QCH = QPW // QK
NT = 10240
RT = NT // NS

BLK = 1000


def _proj_body(x_ref, w_ref, b_ref, p_ref, r_ref):
    o = jnp.dot(x_ref[...], w_ref[...], preferred_element_type=jnp.float32)
    o = o + b_ref[...]
    p_ref[...] = o[:, :D_H]
    r_ref[...] = o[:, D_H:]


def _proj(x, W, b):
    d = x.shape[1]
    return pl.pallas_call(
        _proj_body,
        grid=(N // BLK,),
        in_specs=[
            pl.BlockSpec((BLK, d), lambda i: (i, 0)),
            pl.BlockSpec((d, 2 * D_H), lambda i: (0, 0)),
            pl.BlockSpec((1, 2 * D_H), lambda i: (0, 0)),
        ],
        out_specs=[
            pl.BlockSpec((BLK, D_H), lambda i: (i, 0)),
            pl.BlockSpec((BLK, D_H), lambda i: (i, 0)),
        ],
        out_shape=[jax.ShapeDtypeStruct((N, D_H), jnp.float32)] * 2,
    )(x, W, b)


def _rinv_of(c_ref):
    cnt = c_ref[0, :, 0:1] + c_ref[1, :, 0:1]
    return 1.0 / jnp.maximum(cnt, 1.0)


def _comb_mm_body(s_ref, c_ref, r_ref, w_ref, b_ref, p2_ref, r2_ref):
    s = s_ref[0] + s_ref[1]
    h = jnp.maximum(s * _rinv_of(c_ref) + r_ref[...], 0.0)
    o = jnp.dot(h, w_ref[...], preferred_element_type=jnp.float32) + b_ref[...]
    p2_ref[...] = o[:, :D_H]
    r2_ref[...] = o[:, D_H:]


def _comb_mm(S, C, r, W, b):
    return pl.pallas_call(
        _comb_mm_body,
        grid=(N // BLK,),
        in_specs=[
            pl.BlockSpec((2, BLK, D_H), lambda i: (0, i, 0)),
            pl.BlockSpec((2, BLK, 16), lambda i: (0, i, 0)),
            pl.BlockSpec((BLK, D_H), lambda i: (i, 0)),
            pl.BlockSpec((D_H, 2 * D_H), lambda i: (0, 0)),
            pl.BlockSpec((1, 2 * D_H), lambda i: (0, 0)),
        ],
        out_specs=[
            pl.BlockSpec((BLK, D_H), lambda i: (i, 0)),
            pl.BlockSpec((BLK, D_H), lambda i: (i, 0)),
        ],
        out_shape=[jax.ShapeDtypeStruct((N, D_H), jnp.float32)] * 2,
    )(S, C, r, W, b)


def _comb_body(s_ref, c_ref, r_ref, z_ref):
    s = s_ref[0] + s_ref[1]
    z_ref[...] = jnp.maximum(s * _rinv_of(c_ref) + r_ref[...], 0.0)


def _comb(S, C, r):
    return pl.pallas_call(
        _comb_body,
        grid=(N // BLK,),
        in_specs=[
            pl.BlockSpec((2, BLK, D_H), lambda i: (0, i, 0)),
            pl.BlockSpec((2, BLK, 16), lambda i: (0, i, 0)),
            pl.BlockSpec((BLK, D_H), lambda i: (i, 0)),
        ],
        out_specs=pl.BlockSpec((BLK, D_H), lambda i: (i, 0)),
        out_shape=jax.ShapeDtypeStruct((N, D_H), jnp.float32),
    )(S, C, r)


@functools.lru_cache(maxsize=None)
def _sc_mesh():
    return plsc.VectorSubcoreMesh(core_axis_name="c", subcore_axis_name="s",
                                  num_cores=NC, num_subcores=NS)


def _agg_body(with_counts, *refs):
    if with_counts:
        (p_hbm, src_hbm, dst_hbm, z64_hbm, z16_hbm, outS_hbm, outC_hbm,
         sharedS, sharedC, idx_s, idx_d, rows, ones, sem) = refs
    else:
        (p_hbm, src_hbm, dst_hbm, z64_hbm, outS_hbm,
         sharedS, idx_s, idx_d, rows, sem) = refs
    cid = lax.axis_index("c")
    sid = lax.axis_index("s")
    wid = cid * NS + sid
    pltpu.sync_copy(z64_hbm.at[pl.ds(sid * RT, RT)],
                    sharedS.at[pl.ds(sid * RT, RT)])
    if with_counts:
        pltpu.sync_copy(z16_hbm.at[pl.ds(sid * RT, RT)],
                        sharedC.at[pl.ds(sid * RT, RT)])

        def _fill(i, _):
            ones[i, :] = jnp.full((L,), 1.0, jnp.float32)
            return 0

        lax.fori_loop(0, EK, _fill, 0)
    plsc.subcore_barrier()

    ebase = wid * EPW

    def _chunk(t, _):
        off = ebase + t * EK
        pltpu.sync_copy(src_hbm.at[pl.ds(off, EK)], idx_s)
        pltpu.sync_copy(dst_hbm.at[pl.ds(off, EK)], idx_d)
        pltpu.async_copy(p_hbm.at[idx_s], rows, sem).wait()
        pltpu.sync_copy(rows, sharedS.at[idx_d], add=True)
        if with_counts:
            pltpu.sync_copy(ones, sharedC.at[idx_d], add=True)
        return 0

    lax.fori_loop(0, ECH, _chunk, 0)
    plsc.subcore_barrier()
    pltpu.sync_copy(sharedS.at[pl.ds(sid * RT, RT)],
                    outS_hbm.at[cid, pl.ds(sid * RT, RT)])
    if with_counts:
        pltpu.sync_copy(sharedC.at[pl.ds(sid * RT, RT)],
                        outC_hbm.at[cid, pl.ds(sid * RT, RT)])


@functools.lru_cache(maxsize=None)
def _agg_counts():
    return pl.kernel(
        functools.partial(_agg_body, True),
        out_type=[jax.ShapeDtypeStruct((NC, NT, D_H), jnp.float32),
                  jax.ShapeDtypeStruct((NC, NT, 16), jnp.float32)],
        mesh=_sc_mesh(),
        compiler_params=pltpu.CompilerParams(use_tc_tiling_on_sc=False,
                                             needs_layout_passes=False),
        scratch_types=[
            pltpu.VMEM_SHARED((NT, D_H), jnp.float32),
            pltpu.VMEM_SHARED((NT, 16), jnp.float32),
            pltpu.VMEM((EK,), jnp.int32),
            pltpu.VMEM((EK,), jnp.int32),
            pltpu.VMEM((EK, D_H), jnp.float32),
            pltpu.VMEM((EK, 16), jnp.float32),
            pltpu.SemaphoreType.DMA,
        ],
    )


@functools.lru_cache(maxsize=None)
def _agg_plain():
    return pl.kernel(
        functools.partial(_agg_body, False),
        out_type=jax.ShapeDtypeStruct((NC, NT, D_H), jnp.float32),
        mesh=_sc_mesh(),
        compiler_params=pltpu.CompilerParams(use_tc_tiling_on_sc=False,
                                             needs_layout_passes=False),
        scratch_types=[
            pltpu.VMEM_SHARED((NT, D_H), jnp.float32),
            pltpu.VMEM((EK,), jnp.int32),
            pltpu.VMEM((EK,), jnp.int32),
            pltpu.VMEM((EK, D_H), jnp.float32),
            pltpu.SemaphoreType.DMA,
        ],
    )


def _dec_body(z_hbm, qs_hbm, qd_hbm, out_hbm, qsv, qdv, rs, rd, ob, sem):
    cid = lax.axis_index("c")
    sid = lax.axis_index("s")
    wid = cid * NS + sid
    qbase = wid * QPW
    lane = lax.iota(jnp.int32, 16)

    def _chunk(t, _):
        off = qbase + t * QK
        pltpu.sync_copy(qs_hbm.at[pl.ds(off, QK)], qsv)
        pltpu.sync_copy(qd_hbm.at[pl.ds(off, QK)], qdv)
        pltpu.async_copy(z_hbm.at[qsv], rs, sem).wait()
        pltpu.async_copy(z_hbm.at[qdv], rd, sem).wait()

        def _group(g, _):
            rowi = g * L + lane

            def _col(j, acc):
                ci = jnp.full((L,), j, jnp.int32)
                a = plsc.load_gather(rs, [rowi, ci])
                b = plsc.load_gather(rd, [rowi, ci])
                return acc + a * b

            acc = lax.fori_loop(0, D_H, _col, jnp.zeros((L,), jnp.float32))
            ob[pl.ds(g * L, L)] = 1.0 / (1.0 + jnp.exp(-acc))
            return 0

        lax.fori_loop(0, QK // L, _group, 0)
        pltpu.sync_copy(ob, out_hbm.at[pl.ds(off, QK)])
        return 0

    lax.fori_loop(0, QCH, _chunk, 0)


@functools.lru_cache(maxsize=None)
def _decode():
    return pl.kernel(
        _dec_body,
        out_type=jax.ShapeDtypeStruct((NQ,), jnp.float32),
        mesh=_sc_mesh(),
        compiler_params=pltpu.CompilerParams(use_tc_tiling_on_sc=False,
                                             needs_layout_passes=False),
        scratch_types=[
            pltpu.VMEM((QK,), jnp.int32),
            pltpu.VMEM((QK,), jnp.int32),
            pltpu.VMEM((QK, D_H), jnp.float32),
            pltpu.VMEM((QK, D_H), jnp.float32),
            pltpu.VMEM((QK,), jnp.float32),
            pltpu.SemaphoreType.DMA,
        ],
    )


def kernel(x, edge_index, src, dst, Wl1, Wr1, b1, Wl2, Wr2, b2):
    srcv, dstv = edge_index[0], edge_index[1]
    W1 = jnp.concatenate([Wl1, Wr1], axis=1)
    bias1 = jnp.concatenate([jnp.zeros((D_H,), jnp.float32), b1]).reshape(1, 2 * D_H)
    W2 = jnp.concatenate([Wl2, Wr2], axis=1)
    bias2 = jnp.concatenate([jnp.zeros((D_H,), jnp.float32), b2]).reshape(1, 2 * D_H)
    z64 = jnp.zeros((NT, D_H), jnp.float32)
    z16 = jnp.zeros((NT, 16), jnp.float32)

    p1, r1 = _proj(x, W1, bias1)
    S1, C = _agg_counts()(p1, srcv, dstv, z64, z16)
    p2, r2 = _comb_mm(S1, C, r1, W2, bias2)
    S2 = _agg_plain()(p2, srcv, dstv, z64)
    z = _comb(S2, C, r2)
    return _decode()(z, src, dst)


# speedup vs baseline: 5.4782x; 1.1348x over previous
import functools

import jax
import jax.numpy as jnp
from jax import lax
from jax.experimental import pallas as pl
from jax.experimental.pallas import tpu as pltpu
from jax.experimental.pallas import tpu_sc as plsc

N = 10000
E = 320000
D_IN = 128
D_H = 64
NQ = 65536

NC, NS, L = 2, 16, 16
NW = NC * NS
EPW = E // NW
EK = 128
ECH = 80
EPWP = ECH * EK
ENB = 4
EG = ECH // ENB
QPW = NQ // NW
QK = 128
QCH = QPW // QK
QNB = 4
QG = QCH // QNB
NT = 10240
RT = NT // NS

BLK = 1000


def _proj_body(x_ref, w_ref, b_ref, p_ref, r_ref):
    o = jnp.dot(x_ref[...], w_ref[...], preferred_element_type=jnp.float32)
    o = o + b_ref[...]
    p_ref[...] = o[:, :D_H]
    r_ref[...] = o[:, D_H:]


def _proj(x, W, b):
    d = x.shape[1]
    return pl.pallas_call(
        _proj_body,
        grid=(N // BLK,),
        in_specs=[
            pl.BlockSpec((BLK, d), lambda i: (i, 0)),
            pl.BlockSpec((d, 2 * D_H), lambda i: (0, 0)),
            pl.BlockSpec((1, 2 * D_H), lambda i: (0, 0)),
        ],
        out_specs=[
            pl.BlockSpec((BLK, D_H), lambda i: (i, 0)),
            pl.BlockSpec((BLK, D_H), lambda i: (i, 0)),
        ],
        out_shape=[jax.ShapeDtypeStruct((N, D_H), jnp.float32)] * 2,
    )(x, W, b)


def _rinv_of(c_ref):
    cnt = c_ref[0, :, 0:1] + c_ref[1, :, 0:1]
    return 1.0 / jnp.maximum(cnt, 1.0)


def _comb_mm_body(s_ref, c_ref, r_ref, w_ref, b_ref, p2_ref, r2_ref):
    s = s_ref[0] + s_ref[1]
    h = jnp.maximum(s * _rinv_of(c_ref) + r_ref[...], 0.0)
    o = jnp.dot(h, w_ref[...], preferred_element_type=jnp.float32) + b_ref[...]
    p2_ref[...] = o[:, :D_H]
    r2_ref[...] = o[:, D_H:]


def _comb_mm(S, C, r, W, b):
    return pl.pallas_call(
        _comb_mm_body,
        grid=(N // BLK,),
        in_specs=[
            pl.BlockSpec((2, BLK, D_H), lambda i: (0, i, 0)),
            pl.BlockSpec((2, BLK, 16), lambda i: (0, i, 0)),
            pl.BlockSpec((BLK, D_H), lambda i: (i, 0)),
            pl.BlockSpec((D_H, 2 * D_H), lambda i: (0, 0)),
            pl.BlockSpec((1, 2 * D_H), lambda i: (0, 0)),
        ],
        out_specs=[
            pl.BlockSpec((BLK, D_H), lambda i: (i, 0)),
            pl.BlockSpec((BLK, D_H), lambda i: (i, 0)),
        ],
        out_shape=[jax.ShapeDtypeStruct((N, D_H), jnp.float32)] * 2,
    )(S, C, r, W, b)


def _comb_body(s_ref, c_ref, r_ref, z_ref):
    s = s_ref[0] + s_ref[1]
    z_ref[...] = jnp.maximum(s * _rinv_of(c_ref) + r_ref[...], 0.0)


def _comb(S, C, r):
    return pl.pallas_call(
        _comb_body,
        grid=(N // BLK,),
        in_specs=[
            pl.BlockSpec((2, BLK, D_H), lambda i: (0, i, 0)),
            pl.BlockSpec((2, BLK, 16), lambda i: (0, i, 0)),
            pl.BlockSpec((BLK, D_H), lambda i: (i, 0)),
        ],
        out_specs=pl.BlockSpec((BLK, D_H), lambda i: (i, 0)),
        out_shape=jax.ShapeDtypeStruct((N, D_H), jnp.float32),
    )(S, C, r)


@functools.lru_cache(maxsize=None)
def _sc_mesh():
    return plsc.VectorSubcoreMesh(core_axis_name="c", subcore_axis_name="s",
                                  num_cores=NC, num_subcores=NS)


def _agg_body(with_counts, *refs):
    if with_counts:
        (p_hbm, src3_hbm, dst3_hbm, z64_hbm, z16_hbm, outS_hbm, outC_hbm,
         sharedS, sharedC, src_buf, dst_buf, rows, ones,
         sem_g, sem_s, sem_c) = refs
    else:
        (p_hbm, src3_hbm, dst3_hbm, z64_hbm, outS_hbm,
         sharedS, src_buf, dst_buf, rows, sem_g, sem_s) = refs
    cid = lax.axis_index("c")
    sid = lax.axis_index("s")
    wid = cid * NS + sid
    pltpu.sync_copy(z64_hbm.at[pl.ds(sid * RT, RT)],
                    sharedS.at[pl.ds(sid * RT, RT)])
    pltpu.sync_copy(src3_hbm.at[wid], src_buf)
    pltpu.sync_copy(dst3_hbm.at[wid], dst_buf)
    if with_counts:
        pltpu.sync_copy(z16_hbm.at[pl.ds(sid * RT, RT)],
                        sharedC.at[pl.ds(sid * RT, RT)])

        def _fill(i, _):
            ones[i, :] = jnp.full((L,), 1.0, jnp.float32)
            return 0

        lax.fori_loop(0, EK, _fill, 0)
    plsc.subcore_barrier()

    def _egroup(g, _):
        gd = []
        for b in range(ENB):
            t = g * ENB + b
            gd.append(pltpu.async_copy(p_hbm.at[src_buf.at[t]], rows.at[b],
                                       sem_g.at[b]))
        sd = []
        for b in range(ENB):
            gd[b].wait()
            t = g * ENB + b
            sd.append(pltpu.async_copy(rows.at[b], sharedS.at[dst_buf.at[t]],
                                       sem_s.at[b], add=True))
            if with_counts:
                sd.append(pltpu.async_copy(ones, sharedC.at[dst_buf.at[t]],
                                           sem_c.at[b], add=True))
        for d in sd:
            d.wait()
        return 0

    lax.fori_loop(0, EG, _egroup, 0)
    plsc.subcore_barrier()
    pltpu.sync_copy(sharedS.at[pl.ds(sid * RT, RT)],
                    outS_hbm.at[cid, pl.ds(sid * RT, RT)])
    if with_counts:
        pltpu.sync_copy(sharedC.at[pl.ds(sid * RT, RT)],
                        outC_hbm.at[cid, pl.ds(sid * RT, RT)])


@functools.lru_cache(maxsize=None)
def _agg_counts():
    return pl.kernel(
        functools.partial(_agg_body, True),
        out_type=[jax.ShapeDtypeStruct((NC, NT, D_H), jnp.float32),
                  jax.ShapeDtypeStruct((NC, NT, 16), jnp.float32)],
        mesh=_sc_mesh(),
        compiler_params=pltpu.CompilerParams(use_tc_tiling_on_sc=False,
                                             needs_layout_passes=False),
        scratch_types=[
            pltpu.VMEM_SHARED((NT, D_H), jnp.float32),
            pltpu.VMEM_SHARED((NT, 16), jnp.float32),
            pltpu.VMEM((ECH, EK), jnp.int32),
            pltpu.VMEM((ECH, EK), jnp.int32),
            pltpu.VMEM((ENB, EK, D_H), jnp.float32),
            pltpu.VMEM((EK, 16), jnp.float32),
            pltpu.SemaphoreType.DMA((ENB,)),
            pltpu.SemaphoreType.DMA((ENB,)),
            pltpu.SemaphoreType.DMA((ENB,)),
        ],
    )


@functools.lru_cache(maxsize=None)
def _agg_plain():
    return pl.kernel(
        functools.partial(_agg_body, False),
        out_type=jax.ShapeDtypeStruct((NC, NT, D_H), jnp.float32),
        mesh=_sc_mesh(),
        compiler_params=pltpu.CompilerParams(use_tc_tiling_on_sc=False,
                                             needs_layout_passes=False),
        scratch_types=[
            pltpu.VMEM_SHARED((NT, D_H), jnp.float32),
            pltpu.VMEM((ECH, EK), jnp.int32),
            pltpu.VMEM((ECH, EK), jnp.int32),
            pltpu.VMEM((ENB, EK, D_H), jnp.float32),
            pltpu.SemaphoreType.DMA((ENB,)),
            pltpu.SemaphoreType.DMA((ENB,)),
        ],
    )


def _dec_body(z_hbm, qs3_hbm, qd3_hbm, out_hbm,
              qs_buf, qd_buf, rs, rd, ob, sem_a, sem_b):
    cid = lax.axis_index("c")
    sid = lax.axis_index("s")
    wid = cid * NS + sid
    qbase = wid * QPW
    lane = lax.iota(jnp.int32, 16)
    pltpu.sync_copy(qs3_hbm.at[wid], qs_buf)
    pltpu.sync_copy(qd3_hbm.at[wid], qd_buf)

    def _qgroup(g, _):
        gd = []
        for b in range(QNB):
            t = g * QNB + b
            gd.append(pltpu.async_copy(z_hbm.at[qs_buf.at[t]], rs.at[b],
                                       sem_a.at[b]))
            gd.append(pltpu.async_copy(z_hbm.at[qd_buf.at[t]], rd.at[b],
                                       sem_b.at[b]))
        for b in range(QNB):
            gd[2 * b].wait()
            gd[2 * b + 1].wait()
            t = g * QNB + b
            rsb = rs.at[b]
            rdb = rd.at[b]

            def _qsub(q, _):
                rowi = q * L + lane
                acc = jnp.zeros((L,), jnp.float32)
                for j in range(D_H):
                    ci = jnp.full((L,), j, jnp.int32)
                    acc = acc + (plsc.load_gather(rsb, [rowi, ci]) *
                                 plsc.load_gather(rdb, [rowi, ci]))
                ob[pl.ds(q * L, L)] = 1.0 / (1.0 + jnp.exp(-acc))
                return 0

            lax.fori_loop(0, QK // L, _qsub, 0)
            pltpu.sync_copy(ob, out_hbm.at[pl.ds(qbase + t * QK, QK)])
        return 0

    lax.fori_loop(0, QG, _qgroup, 0)


@functools.lru_cache(maxsize=None)
def _decode():
    return pl.kernel(
        _dec_body,
        out_type=jax.ShapeDtypeStruct((NQ,), jnp.float32),
        mesh=_sc_mesh(),
        compiler_params=pltpu.CompilerParams(use_tc_tiling_on_sc=False,
                                             needs_layout_passes=False),
        scratch_types=[
            pltpu.VMEM((QCH, QK), jnp.int32),
            pltpu.VMEM((QCH, QK), jnp.int32),
            pltpu.VMEM((QNB, QK, D_H), jnp.float32),
            pltpu.VMEM((QNB, QK, D_H), jnp.float32),
            pltpu.VMEM((QK,), jnp.float32),
            pltpu.SemaphoreType.DMA((QNB,)),
            pltpu.SemaphoreType.DMA((QNB,)),
        ],
    )


def kernel(x, edge_index, src, dst, Wl1, Wr1, b1, Wl2, Wr2, b2):
    srcv, dstv = edge_index[0], edge_index[1]
    W1 = jnp.concatenate([Wl1, Wr1], axis=1)
    bias1 = jnp.concatenate([jnp.zeros((D_H,), jnp.float32), b1]).reshape(1, 2 * D_H)
    W2 = jnp.concatenate([Wl2, Wr2], axis=1)
    bias2 = jnp.concatenate([jnp.zeros((D_H,), jnp.float32), b2]).reshape(1, 2 * D_H)
    z64 = jnp.zeros((NT, D_H), jnp.float32)
    z16 = jnp.zeros((NT, 16), jnp.float32)

    pad = EPWP - EPW
    src3 = jnp.pad(srcv.reshape(NW, EPW), ((0, 0), (0, pad))
                   ).reshape(NW, ECH, EK)
    dst3 = jnp.pad(dstv.reshape(NW, EPW), ((0, 0), (0, pad)),
                   constant_values=N).reshape(NW, ECH, EK)
    qs3 = src.reshape(NW, QCH, QK)
    qd3 = dst.reshape(NW, QCH, QK)

    p1, r1 = _proj(x, W1, bias1)
    S1, C = _agg_counts()(p1, src3, dst3, z64, z16)
    p2, r2 = _comb_mm(S1, C, r1, W2, bias2)
    S2 = _agg_plain()(p2, src3, dst3, z64)
    z = _comb(S2, C, r2)
    return _decode()(z, qs3, qd3)


# speedup vs baseline: 12.4704x; 2.2764x over previous
import functools

import jax
import jax.numpy as jnp
from jax import lax
from jax.experimental import pallas as pl
from jax.experimental.pallas import tpu as pltpu
from jax.experimental.pallas import tpu_sc as plsc

N = 10000
E = 320000
D_IN = 128
D_H = 64
NQ = 65536

NC, NS, L = 2, 16, 16
NW = NC * NS
EPW = E // NW
EK = 128
ECH = 80
EPWP = ECH * EK
ENB_C = 4
ENB_P = 4
EPH = 5
HCH = ECH // EPH
CW = 8
QPW = NQ // NW
QK = 128
QCH = QPW // QK
QNB = 4
QG = QCH // QNB
NT = 10240
RT = NT // NS

BLK = 1024


def _proj_body(x_ref, w_ref, b_ref, p_ref, r_ref):
    o = jnp.dot(x_ref[...], w_ref[...], preferred_element_type=jnp.float32)
    o = o + b_ref[...]
    p_ref[...] = o[:, :D_H]
    r_ref[...] = o[:, D_H:]


def _proj(x, W, b):
    d = x.shape[1]
    return pl.pallas_call(
        _proj_body,
        grid=(NT // BLK,),
        in_specs=[
            pl.BlockSpec((BLK, d), lambda i: (i, 0)),
            pl.BlockSpec((d, 2 * D_H), lambda i: (0, 0)),
            pl.BlockSpec((1, 2 * D_H), lambda i: (0, 0)),
        ],
        out_specs=[
            pl.BlockSpec((BLK, D_H), lambda i: (i, 0)),
            pl.BlockSpec((BLK, D_H), lambda i: (i, 0)),
        ],
        out_shape=[jax.ShapeDtypeStruct((NT, D_H), jnp.float32)] * 2,
    )(x, W, b)


def _rinv_of(c_ref):
    cnt = c_ref[0, :, 0:1] + c_ref[1, :, 0:1]
    return 1.0 / jnp.maximum(cnt, 1.0)


def _comb_mm_body(s_ref, c_ref, r_ref, w_ref, b_ref, p2_ref, r2_ref):
    s = s_ref[0] + s_ref[1]
    h = jnp.maximum(s * _rinv_of(c_ref) + r_ref[...], 0.0)
    o = jnp.dot(h, w_ref[...], preferred_element_type=jnp.float32) + b_ref[...]
    p2_ref[...] = o[:, :D_H]
    r2_ref[...] = o[:, D_H:]


def _comb_mm(S, C, r, W, b):
    return pl.pallas_call(
        _comb_mm_body,
        grid=(NT // BLK,),
        in_specs=[
            pl.BlockSpec((2, BLK, D_H), lambda i: (0, i, 0)),
            pl.BlockSpec((2, BLK, CW), lambda i: (0, i, 0)),
            pl.BlockSpec((BLK, D_H), lambda i: (i, 0)),
            pl.BlockSpec((D_H, 2 * D_H), lambda i: (0, 0)),
            pl.BlockSpec((1, 2 * D_H), lambda i: (0, 0)),
        ],
        out_specs=[
            pl.BlockSpec((BLK, D_H), lambda i: (i, 0)),
            pl.BlockSpec((BLK, D_H), lambda i: (i, 0)),
        ],
        out_shape=[jax.ShapeDtypeStruct((NT, D_H), jnp.float32)] * 2,
    )(S, C, r, W, b)


def _comb_body(s_ref, c_ref, r_ref, z_ref):
    s = s_ref[0] + s_ref[1]
    z_ref[...] = jnp.maximum(s * _rinv_of(c_ref) + r_ref[...], 0.0)


def _comb(S, C, r):
    return pl.pallas_call(
        _comb_body,
        grid=(NT // BLK,),
        in_specs=[
            pl.BlockSpec((2, BLK, D_H), lambda i: (0, i, 0)),
            pl.BlockSpec((2, BLK, CW), lambda i: (0, i, 0)),
            pl.BlockSpec((BLK, D_H), lambda i: (i, 0)),
        ],
        out_specs=pl.BlockSpec((BLK, D_H), lambda i: (i, 0)),
        out_shape=jax.ShapeDtypeStruct((NT, D_H), jnp.float32),
    )(S, C, r)


@functools.lru_cache(maxsize=None)
def _sc_mesh():
    return plsc.VectorSubcoreMesh(core_axis_name="c", subcore_axis_name="s",
                                  num_cores=NC, num_subcores=NS)


def _agg_body(with_counts, enb, *refs):
    if with_counts:
        (p_hbm, src3_hbm, dst3_hbm, z64_hbm, z16_hbm, ones_hbm,
         outS_hbm, outC_hbm,
         sharedS, sharedC, sharedP, src_buf, dst_buf, rows, ones,
         sem_g, sem_s, sem_c, sem_i) = refs
    else:
        (p_hbm, src3_hbm, dst3_hbm, z64_hbm, outS_hbm,
         sharedS, sharedP, src_buf, dst_buf, rows, sem_g, sem_s,
         sem_i) = refs
    cid = lax.axis_index("c")
    sid = lax.axis_index("s")
    wid = cid * NS + sid
    pltpu.sync_copy(z64_hbm.at[pl.ds(sid * RT, RT)],
                    sharedS.at[pl.ds(sid * RT, RT)])
    pltpu.sync_copy(p_hbm.at[pl.ds(sid * RT, RT)],
                    sharedP.at[pl.ds(sid * RT, RT)])
    if with_counts:
        pltpu.sync_copy(z16_hbm.at[pl.ds(sid * RT, RT)],
                        sharedC.at[pl.ds(sid * RT, RT)])
        pltpu.sync_copy(ones_hbm, ones)
    plsc.subcore_barrier()

    def _egroup(g, _, ph):
        gd = []
        for b in range(enb):
            t = g * enb + b
            gd.append(pltpu.async_copy(sharedP.at[src_buf.at[ph, t]],
                                       rows.at[b], sem_g.at[b]))
        sd = []
        for b in range(enb):
            gd[b].wait()
            t = g * enb + b
            sd.append(pltpu.async_copy(rows.at[b],
                                       sharedS.at[dst_buf.at[ph, t]],
                                       sem_s.at[b], add=True))
            if with_counts:
                sd.append(pltpu.async_copy(ones,
                                           sharedC.at[dst_buf.at[ph, t]],
                                           sem_c.at[b], add=True))
        for d in sd:
            d.wait()
        return 0

    assert HCH % enb == 0
    idx_d = [None] * EPH
    idx_d[0] = [pltpu.async_copy(src3_hbm.at[wid, pl.ds(0, HCH)],
                                 src_buf.at[0], sem_i.at[0]),
                pltpu.async_copy(dst3_hbm.at[wid, pl.ds(0, HCH)],
                                 dst_buf.at[0], sem_i.at[1])]
    for h in range(EPH):
        if h + 1 < EPH:
            hb = (h + 1) % 2
            idx_d[h + 1] = [
                pltpu.async_copy(src3_hbm.at[wid, pl.ds((h + 1) * HCH, HCH)],
                                 src_buf.at[hb], sem_i.at[2 * hb]),
                pltpu.async_copy(dst3_hbm.at[wid, pl.ds((h + 1) * HCH, HCH)],
                                 dst_buf.at[hb], sem_i.at[2 * hb + 1])]
        for d in idx_d[h]:
            d.wait()
        lax.fori_loop(0, HCH // enb,
                      functools.partial(_egroup, ph=h % 2), 0)
    plsc.subcore_barrier()
    pltpu.sync_copy(sharedS.at[pl.ds(sid * RT, RT)],
                    outS_hbm.at[cid, pl.ds(sid * RT, RT)])
    if with_counts:
        pltpu.sync_copy(sharedC.at[pl.ds(sid * RT, RT)],
                        outC_hbm.at[cid, pl.ds(sid * RT, RT)])


@functools.lru_cache(maxsize=None)
def _agg_counts():
    return pl.kernel(
        functools.partial(_agg_body, True, ENB_C),
        out_type=[jax.ShapeDtypeStruct((NC, NT, D_H), jnp.float32),
                  jax.ShapeDtypeStruct((NC, NT, CW), jnp.float32)],
        mesh=_sc_mesh(),
        compiler_params=pltpu.CompilerParams(use_tc_tiling_on_sc=False,
                                             needs_layout_passes=False),
        scratch_types=[
            pltpu.VMEM_SHARED((NT, D_H), jnp.float32),
            pltpu.VMEM_SHARED((NT, CW), jnp.float32),
            pltpu.VMEM_SHARED((NT, D_H), jnp.float32),
            pltpu.VMEM((2, HCH, EK), jnp.int32),
            pltpu.VMEM((2, HCH, EK), jnp.int32),
            pltpu.VMEM((ENB_C, EK, D_H), jnp.float32),
            pltpu.VMEM((EK, CW), jnp.float32),
            pltpu.SemaphoreType.DMA((ENB_C,)),
            pltpu.SemaphoreType.DMA((ENB_C,)),
            pltpu.SemaphoreType.DMA((ENB_C,)),
            pltpu.SemaphoreType.DMA((4,)),
        ],
    )


@functools.lru_cache(maxsize=None)
def _agg_plain():
    return pl.kernel(
        functools.partial(_agg_body, False, ENB_P),
        out_type=jax.ShapeDtypeStruct((NC, NT, D_H), jnp.float32),
        mesh=_sc_mesh(),
        compiler_params=pltpu.CompilerParams(use_tc_tiling_on_sc=False,
                                             needs_layout_passes=False),
        scratch_types=[
            pltpu.VMEM_SHARED((NT, D_H), jnp.float32),
            pltpu.VMEM_SHARED((NT, D_H), jnp.float32),
            pltpu.VMEM((2, HCH, EK), jnp.int32),
            pltpu.VMEM((2, HCH, EK), jnp.int32),
            pltpu.VMEM((ENB_P, EK, D_H), jnp.float32),
            pltpu.SemaphoreType.DMA((ENB_P,)),
            pltpu.SemaphoreType.DMA((ENB_P,)),
            pltpu.SemaphoreType.DMA((4,)),
        ],
    )


def _dec_body(z_hbm, qs3_hbm, qd3_hbm, out_hbm,
              sharedZ, qs_buf, qd_buf, rs, rd, ob, sem_a, sem_b):
    cid = lax.axis_index("c")
    sid = lax.axis_index("s")
    wid = cid * NS + sid
    qbase = wid * QPW
    lane = lax.iota(jnp.int32, 16)
    pltpu.sync_copy(z_hbm.at[pl.ds(sid * RT, RT)],
                    sharedZ.at[pl.ds(sid * RT, RT)])
    pltpu.sync_copy(qs3_hbm.at[wid], qs_buf)
    pltpu.sync_copy(qd3_hbm.at[wid], qd_buf)
    plsc.subcore_barrier()

    def _qgroup(g, _):
        gd = []
        for b in range(QNB):
            t = g * QNB + b
            gd.append(pltpu.async_copy(sharedZ.at[qs_buf.at[t]], rs.at[b],
                                       sem_a.at[b]))
            gd.append(pltpu.async_copy(sharedZ.at[qd_buf.at[t]], rd.at[b],
                                       sem_b.at[b]))
        for b in range(QNB):
            gd[2 * b].wait()
            gd[2 * b + 1].wait()
            t = g * QNB + b
            rsb = rs.at[b]
            rdb = rd.at[b]

            def _qsub(q, _):
                rowi = q * L + lane
                acc = jnp.zeros((L,), jnp.float32)
                for j in range(D_H):
                    ci = (lane + j) & (D_H - 1)
                    acc = acc + (plsc.load_gather(rsb, [rowi, ci]) *
                                 plsc.load_gather(rdb, [rowi, ci]))
                ob[pl.ds(q * L, L)] = 1.0 / (1.0 + jnp.exp(-acc))
                return 0

            lax.fori_loop(0, QK // L, _qsub, 0)
            pltpu.sync_copy(ob, out_hbm.at[pl.ds(qbase + t * QK, QK)])
        return 0

    lax.fori_loop(0, QG, _qgroup, 0)


@functools.lru_cache(maxsize=None)
def _decode():
    return pl.kernel(
        _dec_body,
        out_type=jax.ShapeDtypeStruct((NQ,), jnp.float32),
        mesh=_sc_mesh(),
        compiler_params=pltpu.CompilerParams(use_tc_tiling_on_sc=False,
                                             needs_layout_passes=False),
        scratch_types=[
            pltpu.VMEM_SHARED((NT, D_H), jnp.float32),
            pltpu.VMEM((QCH, QK), jnp.int32),
            pltpu.VMEM((QCH, QK), jnp.int32),
            pltpu.VMEM((QNB, QK, D_H), jnp.float32),
            pltpu.VMEM((QNB, QK, D_H), jnp.float32),
            pltpu.VMEM((QK,), jnp.float32),
            pltpu.SemaphoreType.DMA((QNB,)),
            pltpu.SemaphoreType.DMA((QNB,)),
        ],
    )


def kernel(x, edge_index, src, dst, Wl1, Wr1, b1, Wl2, Wr2, b2):
    srcv, dstv = edge_index[0], edge_index[1]
    W1 = jnp.concatenate([Wl1, Wr1], axis=1)
    bias1 = jnp.concatenate([jnp.zeros((D_H,), jnp.float32), b1]).reshape(1, 2 * D_H)
    W2 = jnp.concatenate([Wl2, Wr2], axis=1)
    bias2 = jnp.concatenate([jnp.zeros((D_H,), jnp.float32), b2]).reshape(1, 2 * D_H)
    z64 = jnp.zeros((NT, D_H), jnp.float32)
    z16 = jnp.zeros((NT, CW), jnp.float32)
    ones = jnp.ones((EK, CW), jnp.float32)

    pad = EPWP - EPW
    src3 = jnp.pad(srcv.reshape(NW, EPW), ((0, 0), (0, pad))
                   ).reshape(NW, ECH, EK)
    dst3 = jnp.pad(dstv.reshape(NW, EPW), ((0, 0), (0, pad)),
                   constant_values=N).reshape(NW, ECH, EK)
    qs3 = src.reshape(NW, QCH, QK)
    qd3 = dst.reshape(NW, QCH, QK)

    p1, r1 = _proj(x, W1, bias1)
    S1, C = _agg_counts()(p1, src3, dst3, z64, z16, ones)
    p2, r2 = _comb_mm(S1, C, r1, W2, bias2)
    S2 = _agg_plain()(p2, src3, dst3, z64)
    z = _comb(S2, C, r2)
    return _decode()(z, qs3, qd3)
